# Initial kernel scaffold; baseline (speedup 1.0000x reference)
import functools

import jax
import jax.numpy as jnp
from jax import lax
from jax.experimental import pallas as pl
from jax.experimental.pallas import tpu as pltpu
from jax.experimental.pallas import tpu_sc as plsc

N = 10000
E = 320000
D = 128
BN_EPS = 1e-5

NC = 2
NS = 16

EDGES_PER_TILE = E // NS
CHUNK = 80
NCHUNK = EDGES_PER_TILE // CHUNK
ROWS_PER_TILE = N // NS


def _sc_scatter(x, edges):
    mesh = plsc.VectorSubcoreMesh(core_axis_name="c", subcore_axis_name="s")

    @functools.partial(
        pl.kernel,
        mesh=mesh,
        out_type=jax.ShapeDtypeStruct((2, N, D), jnp.float32),
        scratch_types=[
            pltpu.VMEM((NCHUNK, CHUNK), jnp.int32),
            pltpu.VMEM((NCHUNK, CHUNK), jnp.int32),
            pltpu.VMEM((CHUNK, D), jnp.float32),
            pltpu.VMEM_SHARED((N, D), jnp.float32),
            pltpu.SemaphoreType.DMA,
        ],
    )
    def scatter_kernel(x_hbm, edges_hbm, out_hbm, src_v, dst_v, rows_v, acc_sh, sem):
        cid = lax.axis_index("c")
        sid = lax.axis_index("s")

        pltpu.sync_copy(edges_hbm.at[cid, 0, sid], src_v)
        pltpu.sync_copy(edges_hbm.at[cid, 1, sid], dst_v)

        r0 = sid * ROWS_PER_TILE
        pltpu.sync_copy(x_hbm.at[pl.ds(r0, ROWS_PER_TILE)],
                        acc_sh.at[pl.ds(r0, ROWS_PER_TILE)])
        plsc.subcore_barrier()

        def body(i, _):
            pltpu.async_copy(x_hbm.at[src_v.at[i]], rows_v, sem).wait()
            pltpu.sync_copy(rows_v, acc_sh.at[dst_v.at[i]], add=True)
            return ()

        lax.fori_loop(0, NCHUNK, body, (), unroll=False)

        plsc.subcore_barrier()
        pltpu.sync_copy(acc_sh.at[pl.ds(r0, ROWS_PER_TILE)],
                        out_hbm.at[cid, pl.ds(r0, ROWS_PER_TILE)])

    return scatter_kernel(x, edges)


def _tc_mlp_body(pre_ref, w1t_ref, w2t_ref, g1_ref, b1_ref, g2_ref, b2_ref, out_ref):
    pre = pre_ref[0]
    h = jnp.dot(pre, w1t_ref[0], preferred_element_type=jnp.float32)
    mean = jnp.mean(h, axis=0, keepdims=True)
    var = jnp.mean((h - mean) * (h - mean), axis=0, keepdims=True)
    h = (h - mean) * lax.rsqrt(var + BN_EPS) * g1_ref[0] + b1_ref[0]
    h = jnp.maximum(h, 0.0)
    h = jnp.dot(h, w2t_ref[0], preferred_element_type=jnp.float32)
    mean = jnp.mean(h, axis=0, keepdims=True)
    var = jnp.mean((h - mean) * (h - mean), axis=0, keepdims=True)
    h = (h - mean) * lax.rsqrt(var + BN_EPS) * g2_ref[0] + b2_ref[0]
    h = jnp.maximum(h, 0.0)

    @pl.when(pl.program_id(0) == 0)
    def _():
        out_ref[...] = h

    @pl.when(pl.program_id(0) == 1)
    def _():
        out_ref[...] += h


def _tc_mlp(pre, w1t, w2t, g1, b1, g2, b2):
    rel_spec = pl.BlockSpec((1, N, D), lambda r: (r, 0, 0))
    w_spec = pl.BlockSpec((1, D, D), lambda r: (r, 0, 0))
    v_spec = pl.BlockSpec((1, 1, D), lambda r: (r, 0, 0))
    return pl.pallas_call(
        _tc_mlp_body,
        grid=(2,),
        in_specs=[rel_spec, w_spec, w_spec, v_spec, v_spec, v_spec, v_spec],
        out_specs=pl.BlockSpec((N, D), lambda r: (0, 0)),
        out_shape=jax.ShapeDtypeStruct((N, D), jnp.float32),
    )(pre, w1t, w2t, g1, b1, g2, b2)


def kernel(x, edge_index_rel0, edge_index_rel1,
           W1_0, W2_0, g1_0, b1_0, g2_0, b2_0,
           W1_1, W2_1, g1_1, b1_1, g2_1, b2_1):
    edges = jnp.stack([edge_index_rel0, edge_index_rel1])
    edges = edges.reshape(2, 2, NS, NCHUNK, CHUNK)
    pre = _sc_scatter(x, edges)

    w1t = jnp.stack([W1_0.T, W1_1.T])
    w2t = jnp.stack([W2_0.T, W2_1.T])
    g1 = jnp.stack([g1_0, g1_1]).reshape(2, 1, D)
    b1 = jnp.stack([b1_0, b1_1]).reshape(2, 1, D)
    g2 = jnp.stack([g2_0, g2_1]).reshape(2, 1, D)
    b2 = jnp.stack([b2_0, b2_1]).reshape(2, 1, D)
    return _tc_mlp(pre, w1t, w2t, g1, b1, g2, b2)


# speedup vs baseline: 4.8399x; 4.8399x over previous
import functools

import jax
import jax.numpy as jnp
from jax import lax
from jax.experimental import pallas as pl
from jax.experimental.pallas import tpu as pltpu
from jax.experimental.pallas import tpu_sc as plsc

N = 10000
E = 320000
D = 128
DH = D // 2
BN_EPS = 1e-5

NC = 2
NS = 16

EDGES_PER_TILE = E // NS
CHUNK = 80
NCHUNK = EDGES_PER_TILE // CHUNK
ROWS_PER_TILE = 624
TAIL_ROWS = N - NS * ROWS_PER_TILE
TAIL_BASE = NS * ROWS_PER_TILE


def _sc_scatter(xa, xb, edges):
    mesh = plsc.VectorSubcoreMesh(core_axis_name="c", subcore_axis_name="s")

    @functools.partial(
        pl.kernel,
        mesh=mesh,
        compiler_params=pltpu.CompilerParams(use_tc_tiling_on_sc=False),
        out_type=jax.ShapeDtypeStruct((2, 2, N, DH), jnp.float32),
        scratch_types=[
            pltpu.VMEM((NCHUNK, CHUNK), jnp.int32),
            pltpu.VMEM((NCHUNK, CHUNK), jnp.int32),
            pltpu.VMEM((CHUNK, DH), jnp.float32),
            pltpu.VMEM_SHARED((N, DH), jnp.float32),
            pltpu.SemaphoreType.DMA,
        ],
    )
    def scatter_kernel(xa_hbm, xb_hbm, edges_hbm, out_hbm,
                       src_v, dst_v, rows_v, acc_sh, sem):
        cid = lax.axis_index("c")
        sid = lax.axis_index("s")

        pltpu.sync_copy(edges_hbm.at[cid, 0, sid], src_v)
        pltpu.sync_copy(edges_hbm.at[cid, 1, sid], dst_v)

        r0 = sid * ROWS_PER_TILE

        for h, xh_hbm in ((0, xa_hbm), (1, xb_hbm)):
            pltpu.sync_copy(xh_hbm.at[pl.ds(r0, ROWS_PER_TILE)],
                            acc_sh.at[pl.ds(r0, ROWS_PER_TILE)])

            @pl.when(sid == NS - 1)
            def _():
                pltpu.sync_copy(xh_hbm.at[pl.ds(TAIL_BASE, TAIL_ROWS)],
                                acc_sh.at[pl.ds(TAIL_BASE, TAIL_ROWS)])

            plsc.subcore_barrier()

            def body(i, _):
                pltpu.async_copy(xh_hbm.at[src_v.at[i]], rows_v, sem).wait()
                pltpu.sync_copy(rows_v, acc_sh.at[dst_v.at[i]], add=True)
                return ()

            lax.fori_loop(0, NCHUNK, body, (), unroll=False)

            plsc.subcore_barrier()
            pltpu.sync_copy(acc_sh.at[pl.ds(r0, ROWS_PER_TILE)],
                            out_hbm.at[cid, h, pl.ds(r0, ROWS_PER_TILE)])

            @pl.when(sid == NS - 1)
            def _():
                pltpu.sync_copy(acc_sh.at[pl.ds(TAIL_BASE, TAIL_ROWS)],
                                out_hbm.at[cid, h, pl.ds(TAIL_BASE, TAIL_ROWS)])

    return scatter_kernel(xa, xb, edges)


def _tc_mlp_body(pre_ref, w1t_ref, w2t_ref, g1_ref, b1_ref, g2_ref, b2_ref, out_ref):
    w1t = w1t_ref[0]
    h = jnp.dot(pre_ref[0, 0], w1t[:DH, :], preferred_element_type=jnp.float32)
    h = h + jnp.dot(pre_ref[0, 1], w1t[DH:, :], preferred_element_type=jnp.float32)
    mean = jnp.mean(h, axis=0, keepdims=True)
    var = jnp.mean((h - mean) * (h - mean), axis=0, keepdims=True)
    h = (h - mean) * lax.rsqrt(var + BN_EPS) * g1_ref[0] + b1_ref[0]
    h = jnp.maximum(h, 0.0)
    h = jnp.dot(h, w2t_ref[0], preferred_element_type=jnp.float32)
    mean = jnp.mean(h, axis=0, keepdims=True)
    var = jnp.mean((h - mean) * (h - mean), axis=0, keepdims=True)
    h = (h - mean) * lax.rsqrt(var + BN_EPS) * g2_ref[0] + b2_ref[0]
    h = jnp.maximum(h, 0.0)

    @pl.when(pl.program_id(0) == 0)
    def _():
        out_ref[...] = h

    @pl.when(pl.program_id(0) == 1)
    def _():
        out_ref[...] += h


def _tc_mlp(pre, w1t, w2t, g1, b1, g2, b2):
    rel_spec = pl.BlockSpec((1, 2, N, DH), lambda r: (r, 0, 0, 0))
    w_spec = pl.BlockSpec((1, D, D), lambda r: (r, 0, 0))
    v_spec = pl.BlockSpec((1, 1, D), lambda r: (r, 0, 0))
    return pl.pallas_call(
        _tc_mlp_body,
        grid=(2,),
        in_specs=[rel_spec, w_spec, w_spec, v_spec, v_spec, v_spec, v_spec],
        out_specs=pl.BlockSpec((N, D), lambda r: (0, 0)),
        out_shape=jax.ShapeDtypeStruct((N, D), jnp.float32),
    )(pre, w1t, w2t, g1, b1, g2, b2)


def kernel(x, edge_index_rel0, edge_index_rel1,
           W1_0, W2_0, g1_0, b1_0, g2_0, b2_0,
           W1_1, W2_1, g1_1, b1_1, g2_1, b2_1):
    edges = jnp.stack([edge_index_rel0, edge_index_rel1])
    edges = edges.reshape(2, 2, NS, NCHUNK, CHUNK)
    xa = x[:, :DH]
    xb = x[:, DH:]
    pre = _sc_scatter(xa, xb, edges)

    w1t = jnp.stack([W1_0.T, W1_1.T])
    w2t = jnp.stack([W2_0.T, W2_1.T])
    g1 = jnp.stack([g1_0, g1_1]).reshape(2, 1, D)
    b1 = jnp.stack([b1_0, b1_1]).reshape(2, 1, D)
    g2 = jnp.stack([g2_0, g2_1]).reshape(2, 1, D)
    b2 = jnp.stack([b2_0, b2_1]).reshape(2, 1, D)
    return _tc_mlp(pre, w1t, w2t, g1, b1, g2, b2)


# speedup vs baseline: 8.0039x; 1.6537x over previous
import functools

import jax
import jax.numpy as jnp
from jax import lax
from jax.experimental import pallas as pl
from jax.experimental.pallas import tpu as pltpu
from jax.experimental.pallas import tpu_sc as plsc

N = 10000
E = 320000
D = 128
DH = D // 2
BN_EPS = 1e-5

NC = 2
NS = 16

EDGES_PER_TILE = E // NS
CHUNK = 80
NCHUNK = EDGES_PER_TILE // CHUNK
ROWS_PER_TILE = 624
TAIL_ROWS = N - NS * ROWS_PER_TILE
TAIL_BASE = NS * ROWS_PER_TILE


def _sc_scatter(xa, xb, edges):
    mesh = plsc.VectorSubcoreMesh(core_axis_name="c", subcore_axis_name="s")

    @functools.partial(
        pl.kernel,
        mesh=mesh,
        compiler_params=pltpu.CompilerParams(use_tc_tiling_on_sc=False),
        out_type=jax.ShapeDtypeStruct((2, 2, N, DH), jnp.float32),
        scratch_types=[
            pltpu.VMEM((NCHUNK, CHUNK), jnp.int32),
            pltpu.VMEM((NCHUNK, CHUNK), jnp.int32),
            pltpu.VMEM((CHUNK, DH), jnp.float32),
            pltpu.VMEM((CHUNK, DH), jnp.float32),
            pltpu.VMEM_SHARED((N, DH), jnp.float32),
            pltpu.SemaphoreType.DMA,
            pltpu.SemaphoreType.DMA,
        ],
    )
    def scatter_kernel(xa_hbm, xb_hbm, edges_hbm, out_hbm,
                       src_v, dst_v, rows0_v, rows1_v, acc_sh, sem0, sem1):
        cid = lax.axis_index("c")
        sid = lax.axis_index("s")

        pltpu.sync_copy(edges_hbm.at[cid, 0, sid], src_v)
        pltpu.sync_copy(edges_hbm.at[cid, 1, sid], dst_v)

        r0 = sid * ROWS_PER_TILE

        for h, xh_hbm in ((0, xa_hbm), (1, xb_hbm)):
            pltpu.sync_copy(xh_hbm.at[pl.ds(r0, ROWS_PER_TILE)],
                            acc_sh.at[pl.ds(r0, ROWS_PER_TILE)])

            @pl.when(sid == NS - 1)
            def _():
                pltpu.sync_copy(xh_hbm.at[pl.ds(TAIL_BASE, TAIL_ROWS)],
                                acc_sh.at[pl.ds(TAIL_BASE, TAIL_ROWS)])

            plsc.subcore_barrier()

            def gather(i, buf, sem):
                pltpu.async_copy(xh_hbm.at[src_v.at[i]], buf, sem)

            def wait_gather(i, buf, sem):
                pltpu.make_async_copy(xh_hbm.at[src_v.at[i]], buf, sem).wait()

            def scatter(i, buf):
                pltpu.sync_copy(buf, acc_sh.at[dst_v.at[i]], add=True)

            gather(0, rows0_v, sem0)

            def body(k, _):
                j0 = 2 * k
                j1 = 2 * k + 1
                gather(j1, rows1_v, sem1)
                wait_gather(j0, rows0_v, sem0)
                scatter(j0, rows0_v)
                gather(jnp.minimum(j0 + 2, NCHUNK - 1), rows0_v, sem0)
                wait_gather(j0, rows1_v, sem1)
                scatter(j1, rows1_v)
                return ()

            lax.fori_loop(0, NCHUNK // 2, body, (), unroll=False)
            wait_gather(NCHUNK - 1, rows0_v, sem0)

            plsc.subcore_barrier()
            pltpu.sync_copy(acc_sh.at[pl.ds(r0, ROWS_PER_TILE)],
                            out_hbm.at[cid, h, pl.ds(r0, ROWS_PER_TILE)])

            @pl.when(sid == NS - 1)
            def _():
                pltpu.sync_copy(acc_sh.at[pl.ds(TAIL_BASE, TAIL_ROWS)],
                                out_hbm.at[cid, h, pl.ds(TAIL_BASE, TAIL_ROWS)])

    return scatter_kernel(xa, xb, edges)


def _tc_mlp_body(pre_ref, w1t_ref, w2t_ref, g1_ref, b1_ref, g2_ref, b2_ref, out_ref):
    w1t = w1t_ref[0]
    h = jnp.dot(pre_ref[0, 0], w1t[:DH, :], preferred_element_type=jnp.float32)
    h = h + jnp.dot(pre_ref[0, 1], w1t[DH:, :], preferred_element_type=jnp.float32)
    mean = jnp.mean(h, axis=0, keepdims=True)
    var = jnp.mean((h - mean) * (h - mean), axis=0, keepdims=True)
    h = (h - mean) * lax.rsqrt(var + BN_EPS) * g1_ref[0] + b1_ref[0]
    h = jnp.maximum(h, 0.0)
    h = jnp.dot(h, w2t_ref[0], preferred_element_type=jnp.float32)
    mean = jnp.mean(h, axis=0, keepdims=True)
    var = jnp.mean((h - mean) * (h - mean), axis=0, keepdims=True)
    h = (h - mean) * lax.rsqrt(var + BN_EPS) * g2_ref[0] + b2_ref[0]
    h = jnp.maximum(h, 0.0)

    @pl.when(pl.program_id(0) == 0)
    def _():
        out_ref[...] = h

    @pl.when(pl.program_id(0) == 1)
    def _():
        out_ref[...] += h


def _tc_mlp(pre, w1t, w2t, g1, b1, g2, b2):
    rel_spec = pl.BlockSpec((1, 2, N, DH), lambda r: (r, 0, 0, 0))
    w_spec = pl.BlockSpec((1, D, D), lambda r: (r, 0, 0))
    v_spec = pl.BlockSpec((1, 1, D), lambda r: (r, 0, 0))
    return pl.pallas_call(
        _tc_mlp_body,
        grid=(2,),
        in_specs=[rel_spec, w_spec, w_spec, v_spec, v_spec, v_spec, v_spec],
        out_specs=pl.BlockSpec((N, D), lambda r: (0, 0)),
        out_shape=jax.ShapeDtypeStruct((N, D), jnp.float32),
    )(pre, w1t, w2t, g1, b1, g2, b2)


def kernel(x, edge_index_rel0, edge_index_rel1,
           W1_0, W2_0, g1_0, b1_0, g2_0, b2_0,
           W1_1, W2_1, g1_1, b1_1, g2_1, b2_1):
    edges = jnp.stack([edge_index_rel0, edge_index_rel1])
    edges = edges.reshape(2, 2, NS, NCHUNK, CHUNK)
    xa = x[:, :DH]
    xb = x[:, DH:]
    pre = _sc_scatter(xa, xb, edges)

    w1t = jnp.stack([W1_0.T, W1_1.T])
    w2t = jnp.stack([W2_0.T, W2_1.T])
    g1 = jnp.stack([g1_0, g1_1]).reshape(2, 1, D)
    b1 = jnp.stack([b1_0, b1_1]).reshape(2, 1, D)
    g2 = jnp.stack([g2_0, g2_1]).reshape(2, 1, D)
    b2 = jnp.stack([b2_0, b2_1]).reshape(2, 1, D)
    return _tc_mlp(pre, w1t, w2t, g1, b1, g2, b2)
